# Initial kernel scaffold; baseline (speedup 1.0000x reference)
import functools

import jax
import jax.numpy as jnp
from jax import lax
from jax.experimental import pallas as pl
from jax.experimental.pallas import tpu as pltpu
from jax.experimental.pallas import tpu_sc as plsc

NUM_ROWS = 100000
DIM = 64
TOTAL = 4096 * 200

_info = plsc.get_sparse_core_info()
NC, NS = _info.num_cores, _info.num_subcores
NW = NC * NS
PER_W = TOTAL // NW
CHUNK = 128
NCHUNK = PER_W // CHUNK


def _add_body(a_ref, b_ref, o_ref):
    o_ref[...] = a_ref[...] + b_ref[...]


def _summed_table(emb, sin):
    rows_blk = 4000
    grid = NUM_ROWS // rows_blk
    spec = pl.BlockSpec((rows_blk, DIM), lambda i: (i, 0))
    return pl.pallas_call(
        _add_body,
        grid=(grid,),
        in_specs=[spec, spec],
        out_specs=spec,
        out_shape=jax.ShapeDtypeStruct((NUM_ROWS, DIM), jnp.float32),
    )(emb, sin)


def _gather_body(table_hbm, idx_hbm, out_hbm, idx_v, rows_v, gsem):
    wid = lax.axis_index("s") * NC + lax.axis_index("c")
    base = wid * PER_W
    pltpu.sync_copy(idx_hbm.at[wid], idx_v)

    def chunk(j, _):
        pltpu.async_copy(table_hbm.at[idx_v.at[j]], rows_v, gsem).wait()
        pltpu.sync_copy(rows_v, out_hbm.at[pl.ds(base + j * CHUNK, CHUNK)])
        return 0

    lax.fori_loop(0, NCHUNK, chunk, 0)


_gather = pl.kernel(
    _gather_body,
    out_type=jax.ShapeDtypeStruct((TOTAL, DIM), jnp.float32),
    mesh=plsc.VectorSubcoreMesh(core_axis_name="c", subcore_axis_name="s"),
    scratch_types=[
        pltpu.VMEM((NCHUNK, CHUNK), jnp.int32),
        pltpu.VMEM((CHUNK, DIM), jnp.float32),
        pltpu.SemaphoreType.DMA,
    ],
)


def kernel(index_tensor, embedding_matrix, sinusoid_table):
    table = _summed_table(embedding_matrix, sinusoid_table)
    idx = index_tensor.astype(jnp.int32).reshape(NW, NCHUNK, CHUNK)
    out = _gather(table, idx)
    return out.reshape(index_tensor.shape + (DIM,))


# speedup vs baseline: 3.1638x; 3.1638x over previous
import functools

import jax
import jax.numpy as jnp
from jax import lax
from jax.experimental import pallas as pl
from jax.experimental.pallas import tpu as pltpu
from jax.experimental.pallas import tpu_sc as plsc

NUM_ROWS = 100000
DIM = 64
TOTAL = 4096 * 200

_info = plsc.get_sparse_core_info()
NC, NS = _info.num_cores, _info.num_subcores
NW = NC * NS
PER_W = TOTAL // NW
CHUNK = 128
NCHUNK = PER_W // CHUNK


def _add_body(a_ref, b_ref, o_ref):
    o_ref[...] = a_ref[...] + b_ref[...]


def _summed_table(emb, sin):
    rows_blk = 4000
    grid = NUM_ROWS // rows_blk
    spec = pl.BlockSpec((rows_blk, DIM), lambda i: (i, 0))
    return pl.pallas_call(
        _add_body,
        grid=(grid,),
        in_specs=[spec, spec],
        out_specs=spec,
        out_shape=jax.ShapeDtypeStruct((NUM_ROWS, DIM), jnp.float32),
    )(emb, sin)


def _gather_body(table_hbm, idx_hbm, out_hbm, idx_v, rows_v, gsem):
    wid = lax.axis_index("s") * NC + lax.axis_index("c")
    base = wid * PER_W
    pltpu.sync_copy(idx_hbm.at[wid], idx_v)

    def chunk(j, _):
        pltpu.async_copy(table_hbm.at[idx_v.at[j]], rows_v, gsem).wait()
        pltpu.sync_copy(rows_v, out_hbm.at[pl.ds(base + j * CHUNK, CHUNK)])
        return 0

    lax.fori_loop(0, NCHUNK, chunk, 0)


_gather = pl.kernel(
    _gather_body,
    out_type=jax.ShapeDtypeStruct((TOTAL, DIM), jnp.float32),
    mesh=plsc.VectorSubcoreMesh(core_axis_name="c", subcore_axis_name="s"),
    scratch_types=[
        pltpu.VMEM((NCHUNK, CHUNK), jnp.int32),
        pltpu.VMEM((CHUNK, DIM), jnp.float32),
        pltpu.SemaphoreType.DMA,
    ],
    compiler_params=pltpu.CompilerParams(use_tc_tiling_on_sc=False),
)


def kernel(index_tensor, embedding_matrix, sinusoid_table):
    table = _summed_table(embedding_matrix, sinusoid_table)
    idx = index_tensor.astype(jnp.int32).reshape(NW, NCHUNK, CHUNK)
    out = _gather(table, idx)
    return out.reshape(index_tensor.shape + (DIM,))


# speedup vs baseline: 3.7061x; 1.1714x over previous
import functools

import jax
import jax.numpy as jnp
from jax import lax
from jax.experimental import pallas as pl
from jax.experimental.pallas import tpu as pltpu
from jax.experimental.pallas import tpu_sc as plsc

NUM_ROWS = 100000
DIM = 64
TOTAL = 4096 * 200

_info = plsc.get_sparse_core_info()
NC, NS = _info.num_cores, _info.num_subcores
NW = NC * NS
PER_W = TOTAL // NW
CHUNK = 128
NCHUNK = PER_W // CHUNK


def _add_body(a_ref, b_ref, o_ref):
    o_ref[...] = a_ref[...] + b_ref[...]


def _summed_table(emb, sin):
    rows_blk = 4000
    grid = NUM_ROWS // rows_blk
    spec = pl.BlockSpec((rows_blk, DIM), lambda i: (i, 0))
    return pl.pallas_call(
        _add_body,
        grid=(grid,),
        in_specs=[spec, spec],
        out_specs=spec,
        out_shape=jax.ShapeDtypeStruct((NUM_ROWS, DIM), jnp.float32),
    )(emb, sin)


SUB = 4
SROWS = SUB * CHUNK
NSUPER = PER_W // SROWS


def _gather_body(table_hbm, idx_hbm, out_hbm, idx_v, rows_v, g0, g1, w0, w1):
    wid = lax.axis_index("s") * NC + lax.axis_index("c")
    base = wid * PER_W
    gsem = (g0, g1)
    wsem = (w0, w1)
    pltpu.sync_copy(idx_hbm.at[wid], idx_v)

    def fire_gather(t, b):
        for g in range(SUB):
            pltpu.async_copy(
                table_hbm.at[idx_v.at[t * SUB + g]],
                rows_v.at[b].at[pl.ds(g * CHUNK, CHUNK)],
                gsem[b],
            )

    def wait_gather(b):
        pltpu.make_async_copy(
            table_hbm.at[pl.ds(0, SROWS)], rows_v.at[b], gsem[b]
        ).wait()

    def fire_wb(t, b):
        pltpu.async_copy(
            rows_v.at[b], out_hbm.at[pl.ds(base + t * SROWS, SROWS)], wsem[b]
        )

    def wait_wb(b):
        pltpu.make_async_copy(
            table_hbm.at[pl.ds(0, SROWS)], rows_v.at[b], wsem[b]
        ).wait()

    fire_gather(0, 0)
    wait_gather(0)
    fire_wb(0, 0)
    fire_gather(1, 1)

    def loop(t2, _):
        for k in range(2):
            t = 1 + 2 * t2 + k
            b = (1 + k) % 2
            wait_gather(b)
            fire_wb(t, b)
            wait_wb(1 - b)
            fire_gather(t + 1, 1 - b)
        return 0

    lax.fori_loop(0, (NSUPER - 2) // 2, loop, 0)

    wait_gather(1)
    fire_wb(NSUPER - 1, 1)
    wait_wb(0)
    wait_wb(1)


_gather = pl.kernel(
    _gather_body,
    out_type=jax.ShapeDtypeStruct((TOTAL, DIM), jnp.float32),
    mesh=plsc.VectorSubcoreMesh(core_axis_name="c", subcore_axis_name="s"),
    scratch_types=[
        pltpu.VMEM((NCHUNK, CHUNK), jnp.int32),
        pltpu.VMEM((2, SROWS, DIM), jnp.float32),
        pltpu.SemaphoreType.DMA,
        pltpu.SemaphoreType.DMA,
        pltpu.SemaphoreType.DMA,
        pltpu.SemaphoreType.DMA,
    ],
    compiler_params=pltpu.CompilerParams(use_tc_tiling_on_sc=False),
)


def kernel(index_tensor, embedding_matrix, sinusoid_table):
    table = _summed_table(embedding_matrix, sinusoid_table)
    idx = index_tensor.astype(jnp.int32).reshape(NW, NCHUNK, CHUNK)
    out = _gather(table, idx)
    return out.reshape(index_tensor.shape + (DIM,))
